# baseline (speedup 1.0000x reference)
import functools

import jax
import jax.numpy as jnp
from jax import lax
from jax.experimental import pallas as pl
from jax.experimental.pallas import tpu as pltpu
from jax.experimental.pallas import tpu_sc as plsc

N = 10000
E = 320000
NFEAT = 128
NCLASS = 32
K = 16

NC = 2
NS = 16
L = 16
FH = NCLASS // NC
EPT = E // NS
BLK = 128
GRP = 4
NBLK = 160
NGRP = NBLK // GRP
EPAD = NBLK * BLK
RPT = N // NS
MROWS = 1000


def _matmul_body(x_ref, w_ref, o_ref):
    o_ref[...] = lax.dot_general(
        x_ref[...], w_ref[...], (((1,), (1,)), ((), ())),
        preferred_element_type=jnp.float32)


def _project(x, w):
    return pl.pallas_call(
        _matmul_body,
        grid=(N // MROWS,),
        in_specs=[
            pl.BlockSpec((MROWS, NFEAT), lambda i: (i, 0)),
            pl.BlockSpec((NCLASS, NFEAT), lambda i: (0, 0)),
        ],
        out_specs=pl.BlockSpec((MROWS, NCLASS), lambda i: (i, 0)),
        out_shape=jax.ShapeDtypeStruct((N, NCLASS), jnp.float32),
    )(x, w)


def _sc_body(y0_hbm, sd_hbm, w_hbm, coef_hbm, bias_hbm, out_hbm,
             src_v, dst_v, w_v, rows3, rows3g, own_v, own_bf,
             coef_v, bias_v, acc, curb,
             gsem0, gsem1, gsem2, ssem0, ssem1, ssem2):
    c = lax.axis_index("c")
    s = lax.axis_index("s")
    r0 = s * RPT
    rows = pl.ds(r0, RPT)
    gsems = (gsem0, gsem1, gsem2)
    ssems = (ssem0, ssem1, ssem2)

    pltpu.sync_copy(sd_hbm.at[s], src_v)
    pltpu.sync_copy(w_hbm.at[s], w_v)
    pltpu.sync_copy(coef_hbm, coef_v)
    pltpu.sync_copy(bias_hbm.at[c], bias_v)

    def _unpack(i, carry):
        j = i // (BLK // L)
        q = i % (BLK // L)
        p = src_v[j, pl.ds(q * L, L)]
        dst_v[j, pl.ds(q * L, L)] = p >> 16
        src_v[j, pl.ds(q * L, L)] = p & 0xFFFF
        return carry
    lax.fori_loop(0, NBLK * (BLK // L), _unpack, 0)

    cvec = coef_v[...]

    def _publish_and_init():
        def _pk(i, carry):
            v = own_v[i]
            own_bf[i] = v.astype(jnp.bfloat16)
            own_v[i] = v * cvec
            return carry
        lax.fori_loop(0, RPT, _pk, 0)
        pltpu.sync_copy(own_bf, curb.at[rows])
        pltpu.sync_copy(own_v, acc.at[rows])

    pltpu.sync_copy(y0_hbm.at[rows, pl.ds(c * FH, FH)], own_bf)

    def _cvt(i, carry):
        own_v[i] = own_bf[i].astype(jnp.float32)
        return carry
    lax.fori_loop(0, RPT, _cvt, 0)
    _publish_and_init()

    def _fire_g(g, u):
        jb = g * GRP
        for b in range(GRP):
            pltpu.async_copy(curb.at[src_v.at[jb + b]],
                             rows3g.at[u, pl.ds(b * BLK, BLK)], gsems[u])

    def _drain_g(g, u):
        jb = g * GRP
        for b in range(GRP):
            pltpu.make_async_copy(curb.at[src_v.at[jb + b]],
                                  rows3g.at[u, pl.ds(b * BLK, BLK)],
                                  gsems[u]).wait()

    def _fire_s(g, u):
        jb = g * GRP
        for b in range(GRP):
            pltpu.async_copy(rows3.at[u, pl.ds(b * BLK, BLK)],
                             acc.at[dst_v.at[jb + b]], ssems[u], add=True)

    def _drain_s(g, u):
        jb = g * GRP
        for b in range(GRP):
            pltpu.make_async_copy(rows3.at[u, pl.ds(b * BLK, BLK)],
                                  acc.at[dst_v.at[jb + b]], ssems[u]).wait()

    def _mult(g, u):
        jb = g * GRP
        gr_u = rows3g.at[u]
        rows_u = rows3.at[u]

        def _mul16(g2, carry):
            j = jb + g2 // (BLK // L)
            q = g2 % (BLK // L)
            wvec = w_v[j, pl.ds(q * L, L)]
            base = g2 * L
            for t in range(L):
                rows_u[base + t] = gr_u[base + t].astype(jnp.float32) * wvec[t]
            return carry
        lax.fori_loop(0, GRP * BLK // L, _mul16, 0)

    def _edge_pass():
        plsc.subcore_barrier()

        _fire_g(0, 0)
        _fire_g(1, 1)
        _drain_g(0, 0)
        _mult(0, 0)
        _fire_s(0, 0)
        _fire_g(2, 2)
        _drain_g(1, 1)
        _mult(1, 1)
        _fire_s(1, 1)

        def _triple(t, carry):
            for uoff in range(3):
                g = 2 + t * 3 + uoff
                u = (2 + uoff) % 3
                un = (u + 1) % 3
                _drain_s(g - 2, un)
                _fire_g(g + 1, un)
                _drain_g(g, u)
                _mult(g, u)
                _fire_s(g, u)
            return carry
        lax.fori_loop(0, (NGRP - 4) // 3, _triple, 0)

        _drain_s(36, 0)
        _fire_g(39, 0)
        _drain_g(38, 2)
        _mult(38, 2)
        _fire_s(38, 2)
        _drain_s(37, 1)
        _drain_g(39, 0)
        _mult(39, 0)
        _fire_s(39, 0)
        _drain_s(38, 2)
        _drain_s(39, 0)
        plsc.subcore_barrier()

    def _step(k, carry):
        _edge_pass()
        pltpu.sync_copy(acc.at[rows], own_v)
        _publish_and_init()
        return carry

    lax.fori_loop(0, K - 1, _step, 0)

    _edge_pass()
    pltpu.sync_copy(acc.at[rows], own_v)
    bvec = bias_v[...]

    def _out(i, carry):
        own_v[i] = own_v[i] + bvec
        return carry
    lax.fori_loop(0, RPT, _out, 0)
    pltpu.sync_copy(own_v, out_hbm.at[rows, pl.ds(c * FH, FH)])


_sc_propagate = pl.kernel(
    _sc_body,
    out_type=jax.ShapeDtypeStruct((N, NCLASS), jnp.float32),
    mesh=plsc.VectorSubcoreMesh(core_axis_name="c", subcore_axis_name="s"),
    compiler_params=pltpu.CompilerParams(use_tc_tiling_on_sc=False),
    scratch_types=[
        pltpu.VMEM((NBLK, BLK), jnp.int32),
        pltpu.VMEM((NBLK, BLK), jnp.int32),
        pltpu.VMEM((NBLK, BLK), jnp.float32),
        pltpu.VMEM((3, GRP * BLK, L), jnp.float32),
        pltpu.VMEM((3, GRP * BLK, FH), jnp.bfloat16),
        pltpu.VMEM((RPT, FH), jnp.float32),
        pltpu.VMEM((RPT, FH), jnp.bfloat16),
        pltpu.VMEM((L,), jnp.float32),
        pltpu.VMEM((FH,), jnp.float32),
        pltpu.VMEM_SHARED((N, FH), jnp.float32),
        pltpu.VMEM_SHARED((N, FH), jnp.bfloat16),
        pltpu.SemaphoreType.DMA,
        pltpu.SemaphoreType.DMA,
        pltpu.SemaphoreType.DMA,
        pltpu.SemaphoreType.DMA,
        pltpu.SemaphoreType.DMA,
        pltpu.SemaphoreType.DMA,
    ],
)


def kernel(x, edge_index, edge_weight, T, W_weight, W_bias):
    delta = (T / K).astype(jnp.float32)

    y0 = _project(x, W_weight).astype(jnp.bfloat16)

    src = edge_index[0].reshape(NS, EPT)
    dst = edge_index[1].reshape(NS, EPT)
    wsc = (edge_weight * delta).reshape(NS, EPT)
    pad = ((0, 0), (0, EPAD - EPT))
    sd3 = jnp.pad((dst << 16) | src, pad).reshape(NS, NBLK, BLK)
    w3 = jnp.pad(wsc, pad).reshape(NS, NBLK, BLK)

    coef = jnp.full((L,), 1.0, jnp.float32) * (1.0 - delta)
    bias2 = W_bias.reshape(NC, FH)

    return _sc_propagate(y0, sd3, w3, coef, bias2)


# speedup vs baseline: 2.6966x; 2.6966x over previous
import functools

import jax
import jax.numpy as jnp
from jax import lax
from jax.experimental import pallas as pl
from jax.experimental.pallas import tpu as pltpu
from jax.experimental.pallas import tpu_sc as plsc

N = 10000
E = 320000
NFEAT = 128
NCLASS = 32
K = 16

NC = 2
NS = 16
L = 16
FH = NCLASS // NC
EPT = E // NS
BLK = 128
GRP = 4
NBLK = 160
NGRP = NBLK // GRP
EPAD = NBLK * BLK
RPT = N // NS
MROWS = 1000


def _matmul_body(x_ref, w_ref, o_ref):
    o_ref[...] = lax.dot_general(
        x_ref[...], w_ref[...], (((1,), (1,)), ((), ())),
        preferred_element_type=jnp.float32)


def _project(x, w):
    return pl.pallas_call(
        _matmul_body,
        grid=(N // MROWS,),
        in_specs=[
            pl.BlockSpec((MROWS, NFEAT), lambda i: (i, 0)),
            pl.BlockSpec((NCLASS, NFEAT), lambda i: (0, 0)),
        ],
        out_specs=pl.BlockSpec((MROWS, NCLASS), lambda i: (i, 0)),
        out_shape=jax.ShapeDtypeStruct((N, NCLASS), jnp.float32),
    )(x, w)


def _sc_body(y0_hbm, src_hbm, dst_hbm, w_hbm, coef_hbm, bias_hbm, out_hbm,
             src_v, dst_v, w_v, rows3, tmp_v, coef_v, bias_v, ya, yb,
             gsem0, gsem1, gsem2, gsem3, ssem0, ssem1, ssem2, ssem3):
    c = lax.axis_index("c")
    s = lax.axis_index("s")
    r0 = s * RPT
    gsems = (gsem0, gsem1, gsem2, gsem3)
    ssems = (ssem0, ssem1, ssem2, ssem3)

    pltpu.sync_copy(src_hbm.at[s], src_v)
    pltpu.sync_copy(dst_hbm.at[s], dst_v)
    pltpu.sync_copy(w_hbm.at[s], w_v)
    pltpu.sync_copy(coef_hbm, coef_v)
    pltpu.sync_copy(bias_hbm.at[c], bias_v)

    pltpu.sync_copy(y0_hbm.at[pl.ds(r0, RPT), pl.ds(c * FH, FH)], tmp_v)
    pltpu.sync_copy(tmp_v, ya.at[pl.ds(r0, RPT)])
    plsc.subcore_barrier()

    def _fire_g(cur, g, u):
        jb = g * GRP
        for b in range(GRP):
            pltpu.async_copy(cur.at[src_v.at[jb + b]],
                             rows3.at[u, pl.ds(b * BLK, BLK)], gsems[u])

    def _drain_g(cur, g, u):
        jb = g * GRP
        for b in range(GRP):
            pltpu.make_async_copy(cur.at[src_v.at[jb + b]],
                                  rows3.at[u, pl.ds(b * BLK, BLK)],
                                  gsems[u]).wait()

    def _fire_s(nxt, g, u):
        jb = g * GRP
        for b in range(GRP):
            pltpu.async_copy(rows3.at[u, pl.ds(b * BLK, BLK)],
                             nxt.at[dst_v.at[jb + b]], ssems[u], add=True)

    def _drain_s(nxt, g, u):
        jb = g * GRP
        for b in range(GRP):
            pltpu.make_async_copy(rows3.at[u, pl.ds(b * BLK, BLK)],
                                  nxt.at[dst_v.at[jb + b]], ssems[u]).wait()

    def _mult(g, u):
        jb = g * GRP
        rows_u = rows3.at[u]

        def _mul16(g2, carry):
            j = jb + g2 // (BLK // L)
            q = g2 % (BLK // L)
            wvec = w_v[j, pl.ds(q * L, L)]
            for t in range(L):
                rows_u[g2 * L + t] = rows_u[g2 * L + t] * wvec[t]
            return carry
        lax.fori_loop(0, GRP * BLK // L, _mul16, 0)

    def _step(cur, nxt):
        pltpu.sync_copy(cur.at[pl.ds(r0, RPT)], tmp_v)
        cvec = coef_v[...]

        def _init(i, carry):
            tmp_v[i] = tmp_v[i] * cvec
            return carry
        lax.fori_loop(0, RPT, _init, 0)
        pltpu.sync_copy(tmp_v, nxt.at[pl.ds(r0, RPT)])
        plsc.subcore_barrier()

        _fire_g(cur, 0, 0)
        _fire_g(cur, 1, 1)
        _fire_g(cur, 2, 2)
        _drain_g(cur, 0, 0)
        _mult(0, 0)
        _fire_s(nxt, 0, 0)
        _fire_g(cur, 3, 3)
        _drain_g(cur, 1, 1)
        _mult(1, 1)
        _fire_s(nxt, 1, 1)

        def _quad(t, carry):
            for uoff in range(4):
                g = 2 + t * 4 + uoff
                u = (2 + uoff) % 4
                _drain_s(nxt, g - 2, (u + 2) % 4)
                _fire_g(cur, g + 2, (u + 2) % 4)
                _drain_g(cur, g, u)
                _mult(g, u)
                _fire_s(nxt, g, u)
            return carry
        lax.fori_loop(0, (NGRP - 4) // 4, _quad, 0)

        _drain_s(nxt, 36, 0)
        _drain_g(cur, 38, 2)
        _mult(38, 2)
        _fire_s(nxt, 38, 2)
        _drain_s(nxt, 37, 1)
        _drain_g(cur, 39, 3)
        _mult(39, 3)
        _fire_s(nxt, 39, 3)
        _drain_s(nxt, 38, 2)
        _drain_s(nxt, 39, 3)
        plsc.subcore_barrier()

    def _two_steps(k, carry):
        _step(ya, yb)
        _step(yb, ya)
        return carry
    lax.fori_loop(0, K // 2, _two_steps, 0)

    pltpu.sync_copy(ya.at[pl.ds(r0, RPT)], tmp_v)
    bvec = bias_v[...]

    def _out(i, carry):
        tmp_v[i] = tmp_v[i] + bvec
        return carry
    lax.fori_loop(0, RPT, _out, 0)
    pltpu.sync_copy(tmp_v, out_hbm.at[pl.ds(r0, RPT), pl.ds(c * FH, FH)])


_sc_propagate = pl.kernel(
    _sc_body,
    out_type=jax.ShapeDtypeStruct((N, NCLASS), jnp.float32),
    mesh=plsc.VectorSubcoreMesh(core_axis_name="c", subcore_axis_name="s"),
    compiler_params=pltpu.CompilerParams(use_tc_tiling_on_sc=False),
    scratch_types=[
        pltpu.VMEM((NBLK, BLK), jnp.int32),
        pltpu.VMEM((NBLK, BLK), jnp.int32),
        pltpu.VMEM((NBLK, BLK), jnp.float32),
        pltpu.VMEM((4, GRP * BLK, L), jnp.float32),
        pltpu.VMEM((RPT, FH), jnp.float32),
        pltpu.VMEM((L,), jnp.float32),
        pltpu.VMEM((FH,), jnp.float32),
        pltpu.VMEM_SHARED((N, FH), jnp.float32),
        pltpu.VMEM_SHARED((N, FH), jnp.float32),
        pltpu.SemaphoreType.DMA,
        pltpu.SemaphoreType.DMA,
        pltpu.SemaphoreType.DMA,
        pltpu.SemaphoreType.DMA,
        pltpu.SemaphoreType.DMA,
        pltpu.SemaphoreType.DMA,
        pltpu.SemaphoreType.DMA,
        pltpu.SemaphoreType.DMA,
    ],
)


def kernel(x, edge_index, edge_weight, T, W_weight, W_bias):
    delta = (T / K).astype(jnp.float32)

    y0 = _project(x, W_weight)

    src = edge_index[0].reshape(NS, EPT)
    dst = edge_index[1].reshape(NS, EPT)
    wsc = (edge_weight * delta).reshape(NS, EPT)
    pad = ((0, 0), (0, EPAD - EPT))
    src3 = jnp.pad(src, pad).reshape(NS, NBLK, BLK)
    dst3 = jnp.pad(dst, pad).reshape(NS, NBLK, BLK)
    w3 = jnp.pad(wsc, pad).reshape(NS, NBLK, BLK)

    coef = jnp.full((L,), 1.0, jnp.float32) * (1.0 - delta)
    bias2 = W_bias.reshape(NC, FH)

    return _sc_propagate(y0, src3, dst3, w3, coef, bias2)


# speedup vs baseline: 2.6975x; 1.0003x over previous
import functools

import jax
import jax.numpy as jnp
from jax import lax
from jax.experimental import pallas as pl
from jax.experimental.pallas import tpu as pltpu
from jax.experimental.pallas import tpu_sc as plsc

N = 10000
E = 320000
NFEAT = 128
NCLASS = 32
K = 16

NC = 2
NS = 16
L = 16
FH = NCLASS // NC
EPT = E // NS
GSZ = 512
NGRP = 40
EPAD = NGRP * GSZ
RPT = N // NS
MROWS = 1000


def _matmul_body(x_ref, w_ref, o_ref):
    o_ref[...] = lax.dot_general(
        x_ref[...], w_ref[...], (((1,), (1,)), ((), ())),
        preferred_element_type=jnp.float32)


def _project(x, w):
    return pl.pallas_call(
        _matmul_body,
        grid=(N // MROWS,),
        in_specs=[
            pl.BlockSpec((MROWS, NFEAT), lambda i: (i, 0)),
            pl.BlockSpec((NCLASS, NFEAT), lambda i: (0, 0)),
        ],
        out_specs=pl.BlockSpec((MROWS, NCLASS), lambda i: (i, 0)),
        out_shape=jax.ShapeDtypeStruct((N, NCLASS), jnp.float32),
    )(x, w)


def _sc_body(y0_hbm, src_hbm, dst_hbm, w_hbm, coef_hbm, bias_hbm, out_hbm,
             src_v, dst_v, w_v, rows3, tmp_v, coef_v, bias_v, ya, yb,
             gsem0, gsem1, gsem2, gsem3, ssem0, ssem1, ssem2, ssem3):
    c = lax.axis_index("c")
    s = lax.axis_index("s")
    r0 = s * RPT
    gsems = (gsem0, gsem1, gsem2, gsem3)
    ssems = (ssem0, ssem1, ssem2, ssem3)

    pltpu.sync_copy(src_hbm.at[s], src_v)
    pltpu.sync_copy(dst_hbm.at[s], dst_v)
    pltpu.sync_copy(w_hbm.at[s], w_v)
    pltpu.sync_copy(coef_hbm, coef_v)
    pltpu.sync_copy(bias_hbm.at[c], bias_v)

    pltpu.sync_copy(y0_hbm.at[pl.ds(r0, RPT), pl.ds(c * FH, FH)], tmp_v)
    pltpu.sync_copy(tmp_v, ya.at[pl.ds(r0, RPT)])
    plsc.subcore_barrier()

    def _fire_g(cur, g, u):
        pltpu.async_copy(cur.at[src_v.at[g]], rows3.at[u], gsems[u])

    def _drain_g(cur, g, u):
        pltpu.make_async_copy(cur.at[src_v.at[g]], rows3.at[u],
                              gsems[u]).wait()

    def _fire_s(nxt, g, u):
        pltpu.async_copy(rows3.at[u], nxt.at[dst_v.at[g]], ssems[u],
                         add=True)

    def _drain_s(nxt, g, u):
        pltpu.make_async_copy(rows3.at[u], nxt.at[dst_v.at[g]],
                              ssems[u]).wait()

    def _mult(g, u):
        rows_u = rows3.at[u]

        def _mul16(g2, carry):
            wvec = w_v[g, pl.ds(g2 * L, L)]
            for t in range(L):
                rows_u[g2 * L + t] = rows_u[g2 * L + t] * wvec[t]
            return carry
        lax.fori_loop(0, GSZ // L, _mul16, 0)

    def _step(cur, nxt):
        pltpu.sync_copy(cur.at[pl.ds(r0, RPT)], tmp_v)
        cvec = coef_v[...]

        def _init(i, carry):
            tmp_v[i] = tmp_v[i] * cvec
            return carry
        lax.fori_loop(0, RPT, _init, 0)
        pltpu.sync_copy(tmp_v, nxt.at[pl.ds(r0, RPT)])
        plsc.subcore_barrier()

        _fire_g(cur, 0, 0)
        _fire_g(cur, 1, 1)
        _fire_g(cur, 2, 2)
        _drain_g(cur, 0, 0)
        _mult(0, 0)
        _fire_s(nxt, 0, 0)
        _fire_g(cur, 3, 3)
        _drain_g(cur, 1, 1)
        _mult(1, 1)
        _fire_s(nxt, 1, 1)

        def _quad(t, carry):
            for uoff in range(4):
                g = 2 + t * 4 + uoff
                u = (2 + uoff) % 4
                _drain_s(nxt, g - 2, (u + 2) % 4)
                _fire_g(cur, g + 2, (u + 2) % 4)
                _drain_g(cur, g, u)
                _mult(g, u)
                _fire_s(nxt, g, u)
            return carry
        lax.fori_loop(0, (NGRP - 4) // 4, _quad, 0)

        _drain_s(nxt, 36, 0)
        _drain_g(cur, 38, 2)
        _mult(38, 2)
        _fire_s(nxt, 38, 2)
        _drain_s(nxt, 37, 1)
        _drain_g(cur, 39, 3)
        _mult(39, 3)
        _fire_s(nxt, 39, 3)
        _drain_s(nxt, 38, 2)
        _drain_s(nxt, 39, 3)
        plsc.subcore_barrier()

    def _two_steps(k, carry):
        _step(ya, yb)
        _step(yb, ya)
        return carry
    lax.fori_loop(0, K // 2, _two_steps, 0)

    pltpu.sync_copy(ya.at[pl.ds(r0, RPT)], tmp_v)
    bvec = bias_v[...]

    def _out(i, carry):
        tmp_v[i] = tmp_v[i] + bvec
        return carry
    lax.fori_loop(0, RPT, _out, 0)
    pltpu.sync_copy(tmp_v, out_hbm.at[pl.ds(r0, RPT), pl.ds(c * FH, FH)])


_sc_propagate = pl.kernel(
    _sc_body,
    out_type=jax.ShapeDtypeStruct((N, NCLASS), jnp.float32),
    mesh=plsc.VectorSubcoreMesh(core_axis_name="c", subcore_axis_name="s"),
    compiler_params=pltpu.CompilerParams(use_tc_tiling_on_sc=False),
    scratch_types=[
        pltpu.VMEM((NGRP, GSZ), jnp.int32),
        pltpu.VMEM((NGRP, GSZ), jnp.int32),
        pltpu.VMEM((NGRP, GSZ), jnp.float32),
        pltpu.VMEM((4, GSZ, L), jnp.float32),
        pltpu.VMEM((RPT, FH), jnp.float32),
        pltpu.VMEM((L,), jnp.float32),
        pltpu.VMEM((FH,), jnp.float32),
        pltpu.VMEM_SHARED((N, FH), jnp.float32),
        pltpu.VMEM_SHARED((N, FH), jnp.float32),
        pltpu.SemaphoreType.DMA,
        pltpu.SemaphoreType.DMA,
        pltpu.SemaphoreType.DMA,
        pltpu.SemaphoreType.DMA,
        pltpu.SemaphoreType.DMA,
        pltpu.SemaphoreType.DMA,
        pltpu.SemaphoreType.DMA,
        pltpu.SemaphoreType.DMA,
    ],
)


def kernel(x, edge_index, edge_weight, T, W_weight, W_bias):
    delta = (T / K).astype(jnp.float32)

    y0 = _project(x, W_weight)

    src = edge_index[0].reshape(NS, EPT)
    dst = edge_index[1].reshape(NS, EPT)
    wsc = (edge_weight * delta).reshape(NS, EPT)
    pad = ((0, 0), (0, EPAD - EPT))
    src3 = jnp.pad(src, pad).reshape(NS, NGRP, GSZ)
    dst3 = jnp.pad(dst, pad).reshape(NS, NGRP, GSZ)
    w3 = jnp.pad(wsc, pad).reshape(NS, NGRP, GSZ)

    coef = jnp.full((L,), 1.0, jnp.float32) * (1.0 - delta)
    bias2 = W_bias.reshape(NC, FH)

    return _sc_propagate(y0, src3, dst3, w3, coef, bias2)
